# baseline (speedup 1.0000x reference)
import functools

import jax
import jax.numpy as jnp
from jax import lax
from jax.experimental import pallas as pl
from jax.experimental.pallas import tpu as pltpu
from jax.experimental.pallas import tpu_sc as plsc

N = 10000
NPAD = 10240
E = 320000
D_IN = 128
D_H = 64
D_OUT = 32
G = 64

NC = 2
NS = 16
NW = NC * NS
CH = 96
NCH = 108
EPT2 = NCH * CH
E2 = NW * EPT2
PADNODE = 10016
RZ = NPAD // NS
SCB = 32
NSC = RZ // SCB

_MESH = dict(core_axis_name="c", subcore_axis_name="s", num_cores=NC,
             num_subcores=NS)


def _rsqrt16(d):
  i = lax.bitcast_convert_type(d, jnp.int32)
  i = 0x5F3759DF - lax.shift_right_arithmetic(i, 1)
  y = lax.bitcast_convert_type(i, jnp.float32)
  half = d * 0.5
  for _ in range(3):
    y = y * (1.5 - half * y * y)
  return y


def _splat(vec_ref, idx):
  return plsc.load_gather(vec_ref, [jnp.full((16,), idx, jnp.int32)])


def _make_deg_kernel(interpret=False):
  mesh = plsc.VectorSubcoreMesh(**_MESH)

  @functools.partial(
      pl.kernel,
      out_type=jax.ShapeDtypeStruct((NC, NPAD), jnp.float32),
      mesh=mesh,
      interpret=interpret,
      compiler_params=pltpu.CompilerParams(use_tc_tiling_on_sc=False,
                                           needs_layout_passes=False),
      scratch_types=[
          pltpu.VMEM((NCH, CH), jnp.int32),
          pltpu.VMEM((CH,), jnp.float32),
          pltpu.VMEM_SHARED((NPAD,), jnp.float32),
          pltpu.SemaphoreType.DMA,
          pltpu.SemaphoreType.DMA,
          pltpu.SemaphoreType.DMA,
          pltpu.SemaphoreType.DMA,
      ],
  )
  def deg_kernel(dst_hbm, zeros1_hbm, out_hbm, didx, ones_v, deg_sh,
                 sem0, sem1, sem2, sem3):
    c = lax.axis_index("c")
    s = lax.axis_index("s")
    wid = s * NC + c
    sems = (sem0, sem1, sem2, sem3)
    rslice = pl.ds(s * RZ, RZ)
    d0 = pltpu.async_copy(dst_hbm.at[wid], didx, sem0)
    d1 = pltpu.async_copy(zeros1_hbm.at[pl.ds(0, RZ)], deg_sh.at[rslice],
                          sem1)
    for v in range(CH // 16):
      ones_v[pl.ds(v * 16, 16)] = jnp.ones((16,), jnp.float32)
    d0.wait()
    d1.wait()
    plsc.subcore_barrier()

    def deg_body(k4, carry):
      for j in range(4):
        pltpu.async_copy(ones_v, deg_sh.at[didx.at[k4 * 4 + j]],
                         sems[j], add=True)
      for j in range(4):
        pltpu.make_async_copy(ones_v, deg_sh.at[didx.at[k4 * 4 + j]],
                              sems[j]).wait()
      return carry

    lax.fori_loop(0, NCH // 4, deg_body, 0)
    plsc.subcore_barrier()
    pltpu.sync_copy(deg_sh.at[rslice], out_hbm.at[c, rslice])

  return deg_kernel


def _make_sc_kernel(D, first_layer, interpret=False):
  mesh = plsc.VectorSubcoreMesh(**_MESH)
  out_type = [jax.ShapeDtypeStruct((NC, NPAD, D), jnp.float32)]
  if first_layer:
    out_type.append(jax.ShapeDtypeStruct((NPAD,), jnp.float32))

  nvec = D // 16

  @functools.partial(
      pl.kernel,
      out_type=out_type,
      mesh=mesh,
      interpret=interpret,
      compiler_params=pltpu.CompilerParams(use_tc_tiling_on_sc=False,
                                           needs_layout_passes=False),
      scratch_types=[
          pltpu.VMEM((EPT2,), jnp.int32),
          pltpu.VMEM((NCH, CH), jnp.int32),
          pltpu.VMEM((CH, D), jnp.float32),
          pltpu.VMEM((CH, D), jnp.float32),
          pltpu.VMEM((CH, D), jnp.float32),
          pltpu.VMEM((CH, D), jnp.float32),
          pltpu.VMEM((RZ,), jnp.float32),
          pltpu.VMEM((RZ,), jnp.float32),
          pltpu.VMEM_SHARED((NPAD, D), jnp.float32),
          pltpu.VMEM_SHARED((NPAD, D), jnp.float32),
          pltpu.SemaphoreType.DMA,
          pltpu.SemaphoreType.DMA,
          pltpu.SemaphoreType.DMA,
          pltpu.SemaphoreType.DMA,
          pltpu.SemaphoreType.DMA,
          pltpu.SemaphoreType.DMA,
          pltpu.SemaphoreType.DMA,
          pltpu.SemaphoreType.DMA,
      ],
  )
  def sc_kernel(*args):
    if first_layer:
      (h_hbm, src_hbm, dst_hbm, zeros_hbm, degp_hbm,
       out_hbm, dinv_hbm,
       sidx, didx, rows0, rows1, rows2, rows3, dinvv, degv, acc, g_sh,
       gsem0, gsem1, gsem2, gsem3, ssem0, ssem1, ssem2, ssem3) = args
    else:
      (h_hbm, src_hbm, dst_hbm, zeros_hbm, dinv_hbm,
       out_hbm,
       sidx, didx, rows0, rows1, rows2, rows3, dinvv, degv, acc, g_sh,
       gsem0, gsem1, gsem2, gsem3, ssem0, ssem1, ssem2, ssem3) = args
    c = lax.axis_index("c")
    s = lax.axis_index("s")
    wid = s * NC + c
    rows = (rows0, rows1, rows2, rows3)
    gsem = (gsem0, gsem1, gsem2, gsem3)
    ssem = (ssem0, ssem1, ssem2, ssem3)
    rslice = pl.ds(s * RZ, RZ)

    descs = [
        pltpu.async_copy(src_hbm.at[pl.ds(wid * EPT2, EPT2)], sidx, gsem0),
        pltpu.async_copy(dst_hbm.at[wid], didx, gsem0),
        pltpu.async_copy(zeros_hbm.at[pl.ds(0, RZ)], acc.at[rslice], gsem0),
        pltpu.async_copy(h_hbm.at[rslice], g_sh.at[rslice], gsem0),
    ]
    if first_layer:
      descs.append(pltpu.async_copy(degp_hbm.at[0, rslice], degv, gsem1))
      descs.append(pltpu.async_copy(degp_hbm.at[1, rslice], dinvv, gsem1))
    else:
      descs.append(pltpu.async_copy(dinv_hbm.at[rslice], dinvv, gsem1))
    for d in descs:
      d.wait()

    if first_layer:
      for r in range(0, RZ, 16):
        ix = pl.ds(r, 16)
        dinvv[ix] = _rsqrt16(degv[ix] + dinvv[ix] + 1.0)

      @pl.when(c == 0)
      def _():
        pltpu.sync_copy(dinvv, dinv_hbm.at[rslice])

    def scale_body(q, carry):
      base = s * RZ + q * SCB
      pltpu.sync_copy(g_sh.at[pl.ds(base, SCB)], rows0.at[pl.ds(0, SCB)])
      for r in range(SCB):
        dsp = _splat(dinvv, q * SCB + r)
        for v in range(nvec):
          rows0[r, pl.ds(v * 16, 16)] = rows0[r, pl.ds(v * 16, 16)] * dsp
      pltpu.sync_copy(rows0.at[pl.ds(0, SCB)], g_sh.at[pl.ds(base, SCB)])
      return carry

    lax.fori_loop(0, NSC, scale_body, 0)
    plsc.subcore_barrier()

    def gather(k, j):
      pltpu.async_copy(g_sh.at[sidx.at[pl.ds(k * CH, CH)]], rows[j], gsem[j])

    def wait_gather(j):
      pltpu.make_async_copy(g_sh.at[sidx.at[pl.ds(0, CH)]], rows[j],
                            gsem[j]).wait()

    def scatter(k, j):
      pltpu.async_copy(rows[j], acc.at[didx.at[k]], ssem[j], add=True)

    def wait_scatter(k, j):
      pltpu.make_async_copy(rows[j], acc.at[didx.at[k]], ssem[j]).wait()

    for j in range(4):
      gather(j, j)

    def body(k4, carry):
      for j in range(4):
        wait_gather(j)
        scatter(k4 * 4 + j, j)
      for j in range(4):
        wait_scatter(k4 * 4 + j, j)
        gather(k4 * 4 + 4 + j, j)
      return carry

    lax.fori_loop(0, NCH // 4 - 1, body, 0)
    for j in range(4):
      wait_gather(j)
      scatter(NCH - 4 + j, j)
    for j in range(4):
      wait_scatter(NCH - 4 + j, j)
    plsc.subcore_barrier()

    def out_body(q, carry):
      base = s * RZ + q * SCB
      pltpu.sync_copy(acc.at[pl.ds(base, SCB)], rows0.at[pl.ds(0, SCB)])

      @pl.when(c == 0)
      def _():
        pltpu.sync_copy(g_sh.at[pl.ds(base, SCB)], rows1.at[pl.ds(0, SCB)])
        for r in range(SCB):
          for v in range(nvec):
            ix = pl.ds(v * 16, 16)
            rows0[r, ix] = rows0[r, ix] + rows1[r, ix]

      for r in range(SCB):
        dsp = _splat(dinvv, q * SCB + r)
        for v in range(nvec):
          ix = pl.ds(v * 16, 16)
          rows0[r, ix] = rows0[r, ix] * dsp
      pltpu.sync_copy(rows0.at[pl.ds(0, SCB)],
                      out_hbm.at[c, pl.ds(base, SCB)])
      return carry

    lax.fori_loop(0, NSC, out_body, 0)

  return sc_kernel


def _tc0_body(x_ref, w_ref, h_ref):
  h_ref[0:N, :] = jnp.dot(x_ref[...], w_ref[...],
                          preferred_element_type=jnp.float32)
  h_ref[N:NPAD, :] = jnp.zeros((NPAD - N, D_H), jnp.float32)


def _tc0(x, W1, interpret=False):
  return pl.pallas_call(
      _tc0_body,
      out_shape=jax.ShapeDtypeStruct((NPAD, D_H), jnp.float32),
      interpret=interpret,
  )(x, W1)


def _tc1_body(r_ref, b1_ref, w2_ref, h2_ref):
  t = r_ref[0] + r_ref[1] + b1_ref[...]
  h2_ref[...] = jnp.dot(jnp.maximum(t, 0.0), w2_ref[...],
                        preferred_element_type=jnp.float32)


def _tc1(r1, b1_2d, W2, interpret=False):
  return pl.pallas_call(
      _tc1_body,
      out_shape=jax.ShapeDtypeStruct((NPAD, D_OUT), jnp.float32),
      interpret=interpret,
  )(r1, b1_2d, W2)


def _tc2_body(r_ref, b2_ref, info_ref, out_ref):
  out2 = r_ref[0, 0:N] + r_ref[1, 0:N] + b2_ref[...]
  gids = lax.broadcasted_iota(jnp.int32, (G, N), 0).astype(jnp.float32)
  onehot = (gids == info_ref[...]).astype(jnp.float32)
  sums = jnp.dot(onehot, out2, preferred_element_type=jnp.float32)
  counts = jnp.sum(onehot, axis=1, keepdims=True)
  out_ref[...] = sums / jnp.maximum(counts, 1.0)


def _tc2(r2, b2_2d, info_f, interpret=False):
  return pl.pallas_call(
      _tc2_body,
      out_shape=jax.ShapeDtypeStruct((G, D_OUT), jnp.float32),
      interpret=interpret,
  )(r2, b2_2d, info_f)


def kernel(x, edge_index, info_batch, W1, b1, W2, b2):
  pad = jnp.full((E2 - E,), PADNODE, dtype=jnp.int32)
  srcp = jnp.concatenate([edge_index[0], pad])
  dst3 = jnp.concatenate([edge_index[1], pad]).reshape(NW, NCH, CH)

  zeros1 = jnp.zeros((RZ,), dtype=jnp.float32)
  zeros_h = jnp.zeros((RZ, D_H), dtype=jnp.float32)
  zeros_o = jnp.zeros((RZ, D_OUT), dtype=jnp.float32)

  degp = _make_deg_kernel()(dst3, zeros1)
  h1 = _tc0(x, W1)
  r1, dinv = _make_sc_kernel(D_H, True)(h1, srcp, dst3, zeros_h, degp)
  h2 = _tc1(r1, b1.reshape(1, D_H), W2)
  (r2,) = _make_sc_kernel(D_OUT, False)(h2, srcp, dst3, zeros_o, dinv)
  out = _tc2(r2, b2.reshape(1, D_OUT),
             info_batch.astype(jnp.float32).reshape(1, N))
  return out


# speedup vs baseline: 1.1013x; 1.1013x over previous
import functools

import jax
import jax.numpy as jnp
from jax import lax
from jax.experimental import pallas as pl
from jax.experimental.pallas import tpu as pltpu
from jax.experimental.pallas import tpu_sc as plsc

N = 10000
NPAD = 10240
E = 320000
D_IN = 128
D_H = 64
D_OUT = 32
G = 64

NC = 2
NS = 16
NW = NC * NS
CH = 128
NCH = 80
EPT2 = NCH * CH
E2 = NW * EPT2
PADNODE = 10016
RZ = NPAD // NS
SCB = 32
NSC = RZ // SCB

_MESH = dict(core_axis_name="c", subcore_axis_name="s", num_cores=NC,
             num_subcores=NS)


def _rsqrt16(d):
  i = lax.bitcast_convert_type(d, jnp.int32)
  i = 0x5F3759DF - lax.shift_right_arithmetic(i, 1)
  y = lax.bitcast_convert_type(i, jnp.float32)
  half = d * 0.5
  for _ in range(3):
    y = y * (1.5 - half * y * y)
  return y


def _splat(vec_ref, idx):
  return plsc.load_gather(vec_ref, [jnp.full((16,), idx, jnp.int32)])


def _make_deg_kernel(interpret=False):
  mesh = plsc.VectorSubcoreMesh(**_MESH)

  @functools.partial(
      pl.kernel,
      out_type=jax.ShapeDtypeStruct((NC, NPAD), jnp.float32),
      mesh=mesh,
      interpret=interpret,
      compiler_params=pltpu.CompilerParams(use_tc_tiling_on_sc=False,
                                           needs_layout_passes=False),
      scratch_types=[
          pltpu.VMEM((NCH, CH), jnp.int32),
          pltpu.VMEM((CH,), jnp.float32),
          pltpu.VMEM_SHARED((NPAD,), jnp.float32),
          pltpu.SemaphoreType.DMA,
          pltpu.SemaphoreType.DMA,
          pltpu.SemaphoreType.DMA,
          pltpu.SemaphoreType.DMA,
      ],
  )
  def deg_kernel(dst_hbm, zeros1_hbm, out_hbm, didx, ones_v, deg_sh,
                 sem0, sem1, sem2, sem3):
    c = lax.axis_index("c")
    s = lax.axis_index("s")
    wid = s * NC + c
    sems = (sem0, sem1, sem2, sem3)
    rslice = pl.ds(s * RZ, RZ)
    d0 = pltpu.async_copy(dst_hbm.at[wid], didx, sem0)
    d1 = pltpu.async_copy(zeros1_hbm.at[pl.ds(0, RZ)], deg_sh.at[rslice],
                          sem1)
    for v in range(CH // 16):
      ones_v[pl.ds(v * 16, 16)] = jnp.ones((16,), jnp.float32)
    d0.wait()
    d1.wait()
    plsc.subcore_barrier()

    def deg_body(k4, carry):
      for j in range(4):
        pltpu.async_copy(ones_v, deg_sh.at[didx.at[k4 * 4 + j]],
                         sems[j], add=True)
      for j in range(4):
        pltpu.make_async_copy(ones_v, deg_sh.at[didx.at[k4 * 4 + j]],
                              sems[j]).wait()
      return carry

    lax.fori_loop(0, NCH // 4, deg_body, 0)
    plsc.subcore_barrier()
    pltpu.sync_copy(deg_sh.at[rslice], out_hbm.at[c, rslice])

  return deg_kernel


def _make_sc_kernel(D, first_layer, interpret=False):
  mesh = plsc.VectorSubcoreMesh(**_MESH)
  out_type = [jax.ShapeDtypeStruct((NC, NPAD, D), jnp.float32)]
  if first_layer:
    out_type.append(jax.ShapeDtypeStruct((NPAD,), jnp.float32))

  nvec = D // 16

  @functools.partial(
      pl.kernel,
      out_type=out_type,
      mesh=mesh,
      interpret=interpret,
      compiler_params=pltpu.CompilerParams(use_tc_tiling_on_sc=False,
                                           needs_layout_passes=False),
      scratch_types=[
          pltpu.VMEM((EPT2,), jnp.int32),
          pltpu.VMEM((NCH, CH), jnp.int32),
          pltpu.VMEM((CH, D), jnp.float32),
          pltpu.VMEM((CH, D), jnp.float32),
          pltpu.VMEM((RZ,), jnp.float32),
          pltpu.VMEM((RZ,), jnp.float32),
          pltpu.VMEM_SHARED((NPAD, D), jnp.float32),
          pltpu.VMEM_SHARED((NPAD, D), jnp.float32),
          pltpu.SemaphoreType.DMA,
          pltpu.SemaphoreType.DMA,
      ],
  )
  def sc_kernel(*args):
    if first_layer:
      (h_hbm, src_hbm, dst_hbm, zeros_hbm, degp_hbm,
       out_hbm, dinv_hbm,
       sidx, didx, rows0, rows1, dinvv, degv, acc, g_sh,
       gsem0, gsem1) = args
    else:
      (h_hbm, src_hbm, dst_hbm, zeros_hbm, dinv_hbm,
       out_hbm,
       sidx, didx, rows0, rows1, dinvv, degv, acc, g_sh,
       gsem0, gsem1) = args
    c = lax.axis_index("c")
    s = lax.axis_index("s")
    wid = s * NC + c
    rows = (rows0, rows1)
    gsem = (gsem0, gsem1)
    rslice = pl.ds(s * RZ, RZ)

    descs = [
        pltpu.async_copy(src_hbm.at[pl.ds(wid * EPT2, EPT2)], sidx, gsem0),
        pltpu.async_copy(dst_hbm.at[wid], didx, gsem0),
        pltpu.async_copy(zeros_hbm.at[pl.ds(0, RZ)], acc.at[rslice], gsem0),
        pltpu.async_copy(h_hbm.at[rslice], g_sh.at[rslice], gsem0),
    ]
    if first_layer:
      descs.append(pltpu.async_copy(degp_hbm.at[0, rslice], degv, gsem1))
      descs.append(pltpu.async_copy(degp_hbm.at[1, rslice], dinvv, gsem1))
    else:
      descs.append(pltpu.async_copy(dinv_hbm.at[rslice], dinvv, gsem1))
    for d in descs:
      d.wait()

    if first_layer:
      for r in range(0, RZ, 16):
        ix = pl.ds(r, 16)
        dinvv[ix] = _rsqrt16(degv[ix] + dinvv[ix] + 1.0)

      @pl.when(c == 0)
      def _():
        pltpu.sync_copy(dinvv, dinv_hbm.at[rslice])

    def scale_body(q, carry):
      base = s * RZ + q * SCB
      pltpu.sync_copy(g_sh.at[pl.ds(base, SCB)], rows0.at[pl.ds(0, SCB)])
      for r in range(SCB):
        dsp = _splat(dinvv, q * SCB + r)
        for v in range(nvec):
          rows0[r, pl.ds(v * 16, 16)] = rows0[r, pl.ds(v * 16, 16)] * dsp
      pltpu.sync_copy(rows0.at[pl.ds(0, SCB)], g_sh.at[pl.ds(base, SCB)])
      return carry

    lax.fori_loop(0, NSC, scale_body, 0)
    plsc.subcore_barrier()

    def gather(k, j):
      pltpu.async_copy(g_sh.at[sidx.at[pl.ds(k * CH, CH)]], rows[j], gsem[j])

    def consume(k, j, prefetch):
      pltpu.make_async_copy(g_sh.at[sidx.at[pl.ds(0, CH)]], rows[j],
                            gsem[j]).wait()
      pltpu.sync_copy(rows[j], acc.at[didx.at[k]], add=True)
      if prefetch:
        gather(k + 2, j)

    gather(0, 0)
    gather(1, 1)

    def body(k2, carry):
      consume(k2 * 2, 0, True)
      consume(k2 * 2 + 1, 1, True)
      return carry

    lax.fori_loop(0, NCH // 2 - 1, body, 0)
    consume(NCH - 2, 0, False)
    consume(NCH - 1, 1, False)
    plsc.subcore_barrier()

    def out_body(q, carry):
      base = s * RZ + q * SCB
      pltpu.sync_copy(acc.at[pl.ds(base, SCB)], rows0.at[pl.ds(0, SCB)])

      @pl.when(c == 0)
      def _():
        pltpu.sync_copy(g_sh.at[pl.ds(base, SCB)], rows1.at[pl.ds(0, SCB)])
        for r in range(SCB):
          for v in range(nvec):
            ix = pl.ds(v * 16, 16)
            rows0[r, ix] = rows0[r, ix] + rows1[r, ix]

      for r in range(SCB):
        dsp = _splat(dinvv, q * SCB + r)
        for v in range(nvec):
          ix = pl.ds(v * 16, 16)
          rows0[r, ix] = rows0[r, ix] * dsp
      pltpu.sync_copy(rows0.at[pl.ds(0, SCB)],
                      out_hbm.at[c, pl.ds(base, SCB)])
      return carry

    lax.fori_loop(0, NSC, out_body, 0)

  return sc_kernel


def _tc0_body(x_ref, w_ref, h_ref):
  h_ref[0:N, :] = jnp.dot(x_ref[...], w_ref[...],
                          preferred_element_type=jnp.float32)
  h_ref[N:NPAD, :] = jnp.zeros((NPAD - N, D_H), jnp.float32)


def _tc0(x, W1, interpret=False):
  return pl.pallas_call(
      _tc0_body,
      out_shape=jax.ShapeDtypeStruct((NPAD, D_H), jnp.float32),
      interpret=interpret,
  )(x, W1)


def _tc1_body(r_ref, b1_ref, w2_ref, h2_ref):
  t = r_ref[0] + r_ref[1] + b1_ref[...]
  h2_ref[...] = jnp.dot(jnp.maximum(t, 0.0), w2_ref[...],
                        preferred_element_type=jnp.float32)


def _tc1(r1, b1_2d, W2, interpret=False):
  return pl.pallas_call(
      _tc1_body,
      out_shape=jax.ShapeDtypeStruct((NPAD, D_OUT), jnp.float32),
      interpret=interpret,
  )(r1, b1_2d, W2)


def _tc2_body(r_ref, b2_ref, info_ref, out_ref):
  out2 = r_ref[0, 0:N] + r_ref[1, 0:N] + b2_ref[...]
  gids = lax.broadcasted_iota(jnp.int32, (G, N), 0).astype(jnp.float32)
  onehot = (gids == info_ref[...]).astype(jnp.float32)
  sums = jnp.dot(onehot, out2, preferred_element_type=jnp.float32)
  counts = jnp.sum(onehot, axis=1, keepdims=True)
  out_ref[...] = sums / jnp.maximum(counts, 1.0)


def _tc2(r2, b2_2d, info_f, interpret=False):
  return pl.pallas_call(
      _tc2_body,
      out_shape=jax.ShapeDtypeStruct((G, D_OUT), jnp.float32),
      interpret=interpret,
  )(r2, b2_2d, info_f)


def kernel(x, edge_index, info_batch, W1, b1, W2, b2):
  pad = jnp.full((E2 - E,), PADNODE, dtype=jnp.int32)
  srcp = jnp.concatenate([edge_index[0], pad])
  dst3 = jnp.concatenate([edge_index[1], pad]).reshape(NW, NCH, CH)

  zeros1 = jnp.zeros((RZ,), dtype=jnp.float32)
  zeros_h = jnp.zeros((RZ, D_H), dtype=jnp.float32)
  zeros_o = jnp.zeros((RZ, D_OUT), dtype=jnp.float32)

  degp = _make_deg_kernel()(dst3, zeros1)
  h1 = _tc0(x, W1)
  r1, dinv = _make_sc_kernel(D_H, True)(h1, srcp, dst3, zeros_h, degp)
  h2 = _tc1(r1, b1.reshape(1, D_H), W2)
  (r2,) = _make_sc_kernel(D_OUT, False)(h2, srcp, dst3, zeros_o, dinv)
  out = _tc2(r2, b2.reshape(1, D_OUT),
             info_batch.astype(jnp.float32).reshape(1, N))
  return out


# speedup vs baseline: 1.1069x; 1.0051x over previous
import functools

import jax
import jax.numpy as jnp
from jax import lax
from jax.experimental import pallas as pl
from jax.experimental.pallas import tpu as pltpu
from jax.experimental.pallas import tpu_sc as plsc

N = 10000
NPAD = 10240
E = 320000
D_IN = 128
D_H = 64
D_OUT = 32
G = 64

NC = 2
NS = 16
NW = NC * NS
CH = 128
NCH = 80
EPT2 = NCH * CH
E2 = NW * EPT2
PADNODE = 10016
RZ = NPAD // NS
SCB = 64
NSC = RZ // SCB

_MESH = dict(core_axis_name="c", subcore_axis_name="s", num_cores=NC,
             num_subcores=NS)


def _rsqrt16(d):
  i = lax.bitcast_convert_type(d, jnp.int32)
  i = 0x5F3759DF - lax.shift_right_arithmetic(i, 1)
  y = lax.bitcast_convert_type(i, jnp.float32)
  half = d * 0.5
  for _ in range(3):
    y = y * (1.5 - half * y * y)
  return y


def _splat(vec_ref, idx):
  return plsc.load_gather(vec_ref, [jnp.full((16,), idx, jnp.int32)])


def _make_deg_kernel(interpret=False):
  mesh = plsc.VectorSubcoreMesh(**_MESH)

  @functools.partial(
      pl.kernel,
      out_type=jax.ShapeDtypeStruct((NC, NPAD), jnp.float32),
      mesh=mesh,
      interpret=interpret,
      compiler_params=pltpu.CompilerParams(use_tc_tiling_on_sc=False,
                                           needs_layout_passes=False),
      scratch_types=[
          pltpu.VMEM((NCH, CH), jnp.int32),
          pltpu.VMEM((CH,), jnp.float32),
          pltpu.VMEM_SHARED((NPAD,), jnp.float32),
          pltpu.SemaphoreType.DMA,
          pltpu.SemaphoreType.DMA,
          pltpu.SemaphoreType.DMA,
          pltpu.SemaphoreType.DMA,
      ],
  )
  def deg_kernel(dst_hbm, zeros1_hbm, out_hbm, didx, ones_v, deg_sh,
                 sem0, sem1, sem2, sem3):
    c = lax.axis_index("c")
    s = lax.axis_index("s")
    wid = s * NC + c
    sems = (sem0, sem1, sem2, sem3)
    rslice = pl.ds(s * RZ, RZ)
    d0 = pltpu.async_copy(dst_hbm.at[wid], didx, sem0)
    d1 = pltpu.async_copy(zeros1_hbm.at[pl.ds(0, RZ)], deg_sh.at[rslice],
                          sem1)
    for v in range(CH // 16):
      ones_v[pl.ds(v * 16, 16)] = jnp.ones((16,), jnp.float32)
    d0.wait()
    d1.wait()
    plsc.subcore_barrier()

    def deg_body(k4, carry):
      for j in range(4):
        pltpu.async_copy(ones_v, deg_sh.at[didx.at[k4 * 4 + j]],
                         sems[j], add=True)
      for j in range(4):
        pltpu.make_async_copy(ones_v, deg_sh.at[didx.at[k4 * 4 + j]],
                              sems[j]).wait()
      return carry

    lax.fori_loop(0, NCH // 4, deg_body, 0)
    plsc.subcore_barrier()
    pltpu.sync_copy(deg_sh.at[rslice], out_hbm.at[c, rslice])

  return deg_kernel


def _make_sc_kernel(D, first_layer, interpret=False):
  mesh = plsc.VectorSubcoreMesh(**_MESH)
  out_type = [jax.ShapeDtypeStruct((NC, NPAD, D), jnp.float32)]
  if first_layer:
    out_type.append(jax.ShapeDtypeStruct((NPAD,), jnp.float32))

  nvec = D // 16

  @functools.partial(
      pl.kernel,
      out_type=out_type,
      mesh=mesh,
      interpret=interpret,
      compiler_params=pltpu.CompilerParams(use_tc_tiling_on_sc=False,
                                           needs_layout_passes=False),
      scratch_types=[
          pltpu.VMEM((EPT2,), jnp.int32),
          pltpu.VMEM((NCH, CH), jnp.int32),
          pltpu.VMEM((CH, D), jnp.float32),
          pltpu.VMEM((CH, D), jnp.float32),
          pltpu.VMEM((RZ,), jnp.float32),
          pltpu.VMEM((RZ,), jnp.float32),
          pltpu.VMEM_SHARED((NPAD, D), jnp.float32),
          pltpu.VMEM_SHARED((NPAD, D), jnp.float32),
          pltpu.SemaphoreType.DMA,
          pltpu.SemaphoreType.DMA,
      ],
  )
  def sc_kernel(*args):
    if first_layer:
      (h_hbm, src_hbm, dst_hbm, zeros_hbm, degp_hbm,
       out_hbm, dinv_hbm,
       sidx, didx, rows0, rows1, dinvv, degv, acc, g_sh,
       gsem0, gsem1) = args
    else:
      (h_hbm, src_hbm, dst_hbm, zeros_hbm, dinv_hbm,
       out_hbm,
       sidx, didx, rows0, rows1, dinvv, degv, acc, g_sh,
       gsem0, gsem1) = args
    c = lax.axis_index("c")
    s = lax.axis_index("s")
    wid = s * NC + c
    rows = (rows0, rows1)
    gsem = (gsem0, gsem1)
    rslice = pl.ds(s * RZ, RZ)

    descs = [
        pltpu.async_copy(src_hbm.at[pl.ds(wid * EPT2, EPT2)], sidx, gsem0),
        pltpu.async_copy(dst_hbm.at[wid], didx, gsem0),
        pltpu.async_copy(zeros_hbm.at[pl.ds(0, RZ)], acc.at[rslice], gsem0),
        pltpu.async_copy(h_hbm.at[rslice], g_sh.at[rslice], gsem0),
    ]
    if first_layer:
      descs.append(pltpu.async_copy(degp_hbm.at[0, rslice], degv, gsem1))
      descs.append(pltpu.async_copy(degp_hbm.at[1, rslice], dinvv, gsem1))
    else:
      descs.append(pltpu.async_copy(dinv_hbm.at[rslice], dinvv, gsem1))
    for d in descs:
      d.wait()

    if first_layer:
      for r in range(0, RZ, 16):
        ix = pl.ds(r, 16)
        dinvv[ix] = _rsqrt16(degv[ix] + dinvv[ix] + 1.0)

      @pl.when(c == 0)
      def _():
        pltpu.sync_copy(dinvv, dinv_hbm.at[rslice])

    def scale_body(q, carry):
      base = s * RZ + q * SCB
      pltpu.sync_copy(g_sh.at[pl.ds(base, SCB)], rows0.at[pl.ds(0, SCB)])
      for r in range(SCB):
        dsp = _splat(dinvv, q * SCB + r)
        for v in range(nvec):
          rows0[r, pl.ds(v * 16, 16)] = rows0[r, pl.ds(v * 16, 16)] * dsp
      pltpu.sync_copy(rows0.at[pl.ds(0, SCB)], g_sh.at[pl.ds(base, SCB)])
      return carry

    lax.fori_loop(0, NSC, scale_body, 0)
    plsc.subcore_barrier()

    def gather(k, j):
      pltpu.async_copy(g_sh.at[sidx.at[pl.ds(k * CH, CH)]], rows[j], gsem[j])

    def consume(k, j, prefetch):
      pltpu.make_async_copy(g_sh.at[sidx.at[pl.ds(0, CH)]], rows[j],
                            gsem[j]).wait()
      pltpu.sync_copy(rows[j], acc.at[didx.at[k]], add=True)
      if prefetch:
        gather(k + 2, j)

    gather(0, 0)
    gather(1, 1)

    def body(k2, carry):
      consume(k2 * 2, 0, True)
      consume(k2 * 2 + 1, 1, True)
      return carry

    lax.fori_loop(0, NCH // 2 - 1, body, 0)
    consume(NCH - 2, 0, False)
    consume(NCH - 1, 1, False)
    plsc.subcore_barrier()

    def out_body(q, carry):
      base = s * RZ + q * SCB
      pltpu.sync_copy(acc.at[pl.ds(base, SCB)], rows0.at[pl.ds(0, SCB)])

      @pl.when(c == 0)
      def _():
        pltpu.sync_copy(g_sh.at[pl.ds(base, SCB)], rows1.at[pl.ds(0, SCB)])
        for r in range(SCB):
          for v in range(nvec):
            ix = pl.ds(v * 16, 16)
            rows0[r, ix] = rows0[r, ix] + rows1[r, ix]

      for r in range(SCB):
        dsp = _splat(dinvv, q * SCB + r)
        for v in range(nvec):
          ix = pl.ds(v * 16, 16)
          rows0[r, ix] = rows0[r, ix] * dsp
      pltpu.sync_copy(rows0.at[pl.ds(0, SCB)],
                      out_hbm.at[c, pl.ds(base, SCB)])
      return carry

    lax.fori_loop(0, NSC, out_body, 0)

  return sc_kernel


def _tc0_body(x_ref, w_ref, h_ref):
  h_ref[0:N, :] = jnp.dot(x_ref[...], w_ref[...],
                          preferred_element_type=jnp.float32)
  h_ref[N:NPAD, :] = jnp.zeros((NPAD - N, D_H), jnp.float32)


def _tc0(x, W1, interpret=False):
  return pl.pallas_call(
      _tc0_body,
      out_shape=jax.ShapeDtypeStruct((NPAD, D_H), jnp.float32),
      interpret=interpret,
  )(x, W1)


def _tc1_body(r_ref, b1_ref, w2_ref, h2_ref):
  t = r_ref[0] + r_ref[1] + b1_ref[...]
  h2_ref[...] = jnp.dot(jnp.maximum(t, 0.0), w2_ref[...],
                        preferred_element_type=jnp.float32)


def _tc1(r1, b1_2d, W2, interpret=False):
  return pl.pallas_call(
      _tc1_body,
      out_shape=jax.ShapeDtypeStruct((NPAD, D_OUT), jnp.float32),
      interpret=interpret,
  )(r1, b1_2d, W2)


def _tc2_body(r_ref, b2_ref, info_ref, out_ref):
  out2 = r_ref[0, 0:N] + r_ref[1, 0:N] + b2_ref[...]
  gids = lax.broadcasted_iota(jnp.int32, (G, N), 0).astype(jnp.float32)
  onehot = (gids == info_ref[...]).astype(jnp.float32)
  sums = jnp.dot(onehot, out2, preferred_element_type=jnp.float32)
  counts = jnp.sum(onehot, axis=1, keepdims=True)
  out_ref[...] = sums / jnp.maximum(counts, 1.0)


def _tc2(r2, b2_2d, info_f, interpret=False):
  return pl.pallas_call(
      _tc2_body,
      out_shape=jax.ShapeDtypeStruct((G, D_OUT), jnp.float32),
      interpret=interpret,
  )(r2, b2_2d, info_f)


def kernel(x, edge_index, info_batch, W1, b1, W2, b2):
  pad = jnp.full((E2 - E,), PADNODE, dtype=jnp.int32)
  srcp = jnp.concatenate([edge_index[0], pad])
  dst3 = jnp.concatenate([edge_index[1], pad]).reshape(NW, NCH, CH)

  zeros1 = jnp.zeros((RZ,), dtype=jnp.float32)
  zeros_h = jnp.zeros((RZ, D_H), dtype=jnp.float32)
  zeros_o = jnp.zeros((RZ, D_OUT), dtype=jnp.float32)

  degp = _make_deg_kernel()(dst3, zeros1)
  h1 = _tc0(x, W1)
  r1, dinv = _make_sc_kernel(D_H, True)(h1, srcp, dst3, zeros_h, degp)
  h2 = _tc1(r1, b1.reshape(1, D_H), W2)
  (r2,) = _make_sc_kernel(D_OUT, False)(h2, srcp, dst3, zeros_o, dinv)
  out = _tc2(r2, b2.reshape(1, D_OUT),
             info_batch.astype(jnp.float32).reshape(1, N))
  return out


# speedup vs baseline: 1.1293x; 1.0203x over previous
import functools

import jax
import jax.numpy as jnp
from jax import lax
from jax.experimental import pallas as pl
from jax.experimental.pallas import tpu as pltpu
from jax.experimental.pallas import tpu_sc as plsc

N = 10000
NPAD = 10240
E = 320000
D_IN = 128
D_H = 64
D_OUT = 32
G = 64

NC = 2
NS = 16
NW = NC * NS
CH = 128
NCH = 80
EPT2 = NCH * CH
E2 = NW * EPT2
PADNODE = 10016
RZ = NPAD // NS

_MESH = dict(core_axis_name="c", subcore_axis_name="s", num_cores=NC,
             num_subcores=NS)


def _make_deg_kernel(interpret=False):
  mesh = plsc.VectorSubcoreMesh(**_MESH)

  @functools.partial(
      pl.kernel,
      out_type=jax.ShapeDtypeStruct((NC, NPAD, 16), jnp.float32),
      mesh=mesh,
      interpret=interpret,
      compiler_params=pltpu.CompilerParams(use_tc_tiling_on_sc=False),
      scratch_types=[
          pltpu.VMEM((CH, 16), jnp.float32),
          pltpu.VMEM((NCH, CH), jnp.int32),
          pltpu.VMEM_SHARED((NPAD, 16), jnp.float32),
          pltpu.SemaphoreType.DMA,
      ],
  )
  def deg_kernel(dst_hbm, ones_hbm, zeros_hbm, out_hbm, ones_v, didx, acc,
                 dsem):
    c = lax.axis_index("c")
    s = lax.axis_index("s")
    wid = s * NC + c
    descs = [
        pltpu.async_copy(zeros_hbm.at[pl.ds(0, RZ)],
                         acc.at[pl.ds(s * RZ, RZ)], dsem),
        pltpu.async_copy(ones_hbm, ones_v, dsem),
        pltpu.async_copy(dst_hbm.at[wid], didx, dsem),
    ]
    for d in descs:
      d.wait()
    plsc.subcore_barrier()

    def body(k4, carry):
      for j in range(4):
        pltpu.async_copy(ones_v, acc.at[didx.at[k4 * 4 + j]], dsem, add=True)
      for j in range(4):
        pltpu.make_async_copy(ones_v, acc.at[didx.at[k4 * 4 + j]],
                              dsem).wait()
      return carry

    lax.fori_loop(0, NCH // 4, body, 0)
    plsc.subcore_barrier()
    pltpu.sync_copy(acc.at[pl.ds(s * RZ, RZ)],
                    out_hbm.at[c, pl.ds(s * RZ, RZ)])

  return deg_kernel


def _make_scatter_kernel(D, interpret=False):
  mesh = plsc.VectorSubcoreMesh(**_MESH)

  @functools.partial(
      pl.kernel,
      out_type=jax.ShapeDtypeStruct((NC, NPAD, D), jnp.float32),
      mesh=mesh,
      interpret=interpret,
      compiler_params=pltpu.CompilerParams(use_tc_tiling_on_sc=False),
      scratch_types=[
          pltpu.VMEM((EPT2,), jnp.int32),
          pltpu.VMEM((NCH, CH), jnp.int32),
          pltpu.VMEM((CH, D), jnp.float32),
          pltpu.VMEM((CH, D), jnp.float32),
          pltpu.VMEM_SHARED((NPAD, D), jnp.float32),
          pltpu.VMEM_SHARED((NPAD, D), jnp.float32),
          pltpu.SemaphoreType.DMA,
          pltpu.SemaphoreType.DMA,
      ],
  )
  def scat_kernel(g_hbm, src_hbm, dst_hbm, zeros_hbm, out_hbm,
                  sidx, didx, rows0, rows1, acc, g_sh,
                  gsem0, gsem1):
    c = lax.axis_index("c")
    s = lax.axis_index("s")
    wid = s * NC + c
    rows = (rows0, rows1)
    gsem = (gsem0, gsem1)
    prologue = (
        lambda sem: pltpu.async_copy(src_hbm.at[pl.ds(wid * EPT2, EPT2)],
                                     sidx, sem),
        lambda sem: pltpu.async_copy(dst_hbm.at[wid], didx, sem),
        lambda sem: pltpu.async_copy(zeros_hbm.at[pl.ds(0, RZ)],
                                     acc.at[pl.ds(s * RZ, RZ)], sem),
        lambda sem: pltpu.async_copy(g_hbm.at[pl.ds(s * RZ, RZ)],
                                     g_sh.at[pl.ds(s * RZ, RZ)], sem),
    )
    descs = [issue(gsem0) for issue in prologue]
    for d in descs:
      d.wait()
    plsc.subcore_barrier()

    def gather(k, j):
      pltpu.async_copy(g_sh.at[sidx.at[pl.ds(k * CH, CH)]], rows[j], gsem[j])

    def consume(k, j, prefetch):
      pltpu.make_async_copy(g_sh.at[sidx.at[pl.ds(0, CH)]], rows[j],
                            gsem[j]).wait()
      pltpu.sync_copy(rows[j], acc.at[didx.at[k]], add=True)
      if prefetch:
        gather(k + 2, j)

    gather(0, 0)
    gather(1, 1)

    def body(k2, carry):
      consume(k2 * 2, 0, True)
      consume(k2 * 2 + 1, 1, True)
      return carry

    lax.fori_loop(0, NCH // 2 - 1, body, 0)
    consume(NCH - 2, 0, False)
    consume(NCH - 1, 1, False)
    plsc.subcore_barrier()
    pltpu.sync_copy(acc.at[pl.ds(s * RZ, RZ)],
                    out_hbm.at[c, pl.ds(s * RZ, RZ)])

  return scat_kernel


def _tc1_body(x_ref, w_ref, d_ref, g_ref, dinv_ref):
  deg = d_ref[0] + d_ref[1] + 1.0
  dinv = lax.rsqrt(deg)
  h = jnp.dot(x_ref[...], w_ref[...], preferred_element_type=jnp.float32)
  g_ref[0:N, :] = h * dinv[0:N, 0:1]
  g_ref[N:NPAD, :] = jnp.zeros((NPAD - N, D_H), jnp.float32)
  dinv_ref[...] = dinv


def _tc1(x, W1, degp, interpret=False):
  return pl.pallas_call(
      _tc1_body,
      out_shape=[
          jax.ShapeDtypeStruct((NPAD, D_H), jnp.float32),
          jax.ShapeDtypeStruct((NPAD, 16), jnp.float32),
      ],
      interpret=interpret,
  )(x, W1, degp)


def _tc2_body(s_ref, g1_ref, dinv_ref, b1_ref, w2_ref, g2_ref):
  dinv = dinv_ref[:, 0:1]
  t = (s_ref[0] + s_ref[1] + g1_ref[...]) * dinv + b1_ref[...]
  r = jnp.maximum(t, 0.0)
  h2 = jnp.dot(r, w2_ref[...], preferred_element_type=jnp.float32)
  g2_ref[...] = h2 * dinv


def _tc2(s1, g1, dinv16, b1_2d, W2, interpret=False):
  return pl.pallas_call(
      _tc2_body,
      out_shape=jax.ShapeDtypeStruct((NPAD, D_OUT), jnp.float32),
      interpret=interpret,
  )(s1, g1, dinv16, b1_2d, W2)


def _tc3_body(s_ref, g2_ref, dinv_ref, b2_ref, info_ref, out_ref):
  dinv = dinv_ref[0:N, 0:1]
  out2 = (s_ref[0, 0:N] + s_ref[1, 0:N] + g2_ref[0:N]) * dinv + b2_ref[...]
  gids = lax.broadcasted_iota(jnp.int32, (G, N), 0).astype(jnp.float32)
  onehot = (gids == info_ref[...]).astype(jnp.float32)
  sums = jnp.dot(onehot, out2, preferred_element_type=jnp.float32)
  counts = jnp.sum(onehot, axis=1, keepdims=True)
  out_ref[...] = sums / jnp.maximum(counts, 1.0)


def _tc3(s2, g2, dinv16, b2_2d, info_f, interpret=False):
  return pl.pallas_call(
      _tc3_body,
      out_shape=jax.ShapeDtypeStruct((G, D_OUT), jnp.float32),
      interpret=interpret,
  )(s2, g2, dinv16, b2_2d, info_f)


def kernel(x, edge_index, info_batch, W1, b1, W2, b2):
  pad = jnp.full((E2 - E,), PADNODE, dtype=jnp.int32)
  srcp = jnp.concatenate([edge_index[0], pad])
  dst3 = jnp.concatenate([edge_index[1], pad]).reshape(NW, NCH, CH)

  ones16 = jnp.ones((CH, 16), dtype=jnp.float32)
  zeros16 = jnp.zeros((RZ, 16), dtype=jnp.float32)
  zeros_h = jnp.zeros((RZ, D_H), dtype=jnp.float32)
  zeros_o = jnp.zeros((RZ, D_OUT), dtype=jnp.float32)

  degp = _make_deg_kernel()(dst3, ones16, zeros16)
  g1, dinv16 = _tc1(x, W1, degp)
  s1 = _make_scatter_kernel(D_H)(g1, srcp, dst3, zeros_h)
  g2 = _tc2(s1, g1, dinv16, b1.reshape(1, D_H), W2)
  s2 = _make_scatter_kernel(D_OUT)(g2, srcp, dst3, zeros_o)
  out = _tc3(s2, g2, dinv16, b2.reshape(1, D_OUT),
             info_batch.astype(jnp.float32).reshape(1, N))
  return out
